# Initial kernel scaffold; baseline (speedup 1.0000x reference)
import jax
import jax.numpy as jnp
from jax.experimental import pallas as pl


def kernel(x, edge_index, edge_type, rel_embed, w_loop, w_in, w_out, w_rel, loop_rel, bn_gamma, bn_beta):
    raise NotImplementedError("write your pallas kernel here")


# speedup vs baseline: 10.2654x; 10.2654x over previous
import functools

import jax
import jax.numpy as jnp
from jax import lax
from jax.experimental import pallas as pl
from jax.experimental.pallas import tpu as pltpu
from jax.experimental.pallas import tpu_sc as plsc

_NC = 2
_NS = 16
_L = 16

_MESH = dict(core_axis_name="c", subcore_axis_name="s")


def _sc_degree(row2, n_nodes):
    e2 = row2.shape[0]
    e = e2 // 2
    ept = e // _NS
    c = 80
    g = ept // c
    zchunk = 1000

    @functools.partial(
        pl.kernel,
        out_type=jax.ShapeDtypeStruct((2 * n_nodes,), jnp.float32),
        mesh=plsc.VectorSubcoreMesh(**_MESH),
        scratch_types=[
            pltpu.VMEM((c,), jnp.float32),
            pltpu.VMEM((c,), jnp.int32),
            pltpu.VMEM((1024,), jnp.float32),
            pltpu.VMEM_SHARED((n_nodes,), jnp.float32),
        ],
    )
    def deg_kernel(row_hbm, deg_hbm, ones_v, row_v, zero_v, deg_sh):
        core = lax.axis_index("c")
        sub = lax.axis_index("s")
        for i in range(c // _L):
            ones_v[pl.ds(i * _L, _L)] = jnp.ones((_L,), jnp.float32)
        for i in range(1024 // _L):
            zero_v[pl.ds(i * _L, _L)] = jnp.zeros((_L,), jnp.float32)

        @pl.when(sub < n_nodes // zchunk)
        def _():
            pltpu.sync_copy(zero_v.at[pl.ds(0, zchunk)],
                            deg_sh.at[pl.ds(sub * zchunk, zchunk)])

        plsc.subcore_barrier()

        def chunk(i, carry):
            base = core * e + sub * ept + i * c
            pltpu.sync_copy(row_hbm.at[pl.ds(base, c)], row_v)
            pltpu.sync_copy(ones_v, deg_sh.at[row_v], add=True)
            return carry

        lax.fori_loop(0, g, chunk, 0)
        plsc.subcore_barrier()

        @pl.when(sub < n_nodes // zchunk)
        def _():
            pltpu.sync_copy(deg_sh.at[pl.ds(sub * zchunk, zchunk)],
                            zero_v.at[pl.ds(0, zchunk)])
            pltpu.sync_copy(
                zero_v.at[pl.ds(0, zchunk)],
                deg_hbm.at[pl.ds(core * n_nodes + sub * zchunk, zchunk)])

    return deg_kernel(row2)


def _sc_edges(row2, colx2, et2, xs_cat, rel_cat):
    e2 = row2.shape[0]
    e = e2 // 2
    ept = e // _NS
    c = 80
    g = ept // c
    n2, h = xs_cat.shape
    n = n2 // 2
    hl = h // _L
    br = 16
    nb = n // br

    @functools.partial(
        pl.kernel,
        out_type=jax.ShapeDtypeStruct((2 * n, h), jnp.float32),
        mesh=plsc.VectorSubcoreMesh(**_MESH),
        scratch_types=[
            pltpu.VMEM((c,), jnp.int32),
            pltpu.VMEM((c,), jnp.int32),
            pltpu.VMEM((c,), jnp.int32),
            pltpu.VMEM((c, h), jnp.float32),
            pltpu.VMEM((c, h), jnp.float32),
            pltpu.VMEM((c, h), jnp.float32),
            pltpu.VMEM((br, h), jnp.float32),
            pltpu.VMEM_SHARED((n, h), jnp.float32),
            pltpu.SemaphoreType.DMA,
            pltpu.SemaphoreType.DMA,
        ],
    )
    def edge_kernel(row_hbm, col_hbm, et_hbm, xs_hbm, rel_hbm, agg_hbm,
                    row_v, col_v, et_v, xb, rb, pb, zb, agg_sh, sem1, sem2):
        core = lax.axis_index("c")
        sub = lax.axis_index("s")

        def zrow(r, carry):
            for j in range(hl):
                zb[r, pl.ds(j * _L, _L)] = jnp.zeros((_L,), jnp.float32)
            return carry

        lax.fori_loop(0, br, zrow, 0)

        def zblk(k, carry):
            blk = k * _NS + sub
            @pl.when(blk < nb)
            def _():
                pltpu.sync_copy(zb, agg_sh.at[pl.ds(blk * br, br)])
            return carry

        lax.fori_loop(0, (nb + _NS - 1) // _NS, zblk, 0)
        plsc.subcore_barrier()

        def chunk(i, carry):
            base = core * e + sub * ept + i * c
            pltpu.sync_copy(row_hbm.at[pl.ds(base, c)], row_v)
            pltpu.sync_copy(col_hbm.at[pl.ds(base, c)], col_v)
            pltpu.sync_copy(et_hbm.at[pl.ds(base, c)], et_v)
            cx = pltpu.async_copy(xs_hbm.at[col_v], xb, sem1)
            cr = pltpu.async_copy(rel_hbm.at[et_v], rb, sem2)
            cx.wait()
            cr.wait()

            def prod(k, carry2):
                for j in range(hl):
                    sl = pl.ds(j * _L, _L)
                    pb[k, sl] = xb[k, sl] * rb[k, sl]
                return carry2

            lax.fori_loop(0, c, prod, 0)
            pltpu.sync_copy(pb, agg_sh.at[row_v], add=True)
            return carry

        lax.fori_loop(0, g, chunk, 0)
        plsc.subcore_barrier()

        def wblk(k, carry):
            blk = k * _NS + sub
            @pl.when(blk < nb)
            def _():
                pltpu.sync_copy(agg_sh.at[pl.ds(blk * br, br)], zb)
                pltpu.sync_copy(zb, agg_hbm.at[pl.ds(core * n + blk * br, br)])
            return carry

        lax.fori_loop(0, (nb + _NS - 1) // _NS, wblk, 0)

    return edge_kernel(row2, colx2, et2, xs_cat, rel_cat)


def _tc_scale(x, deg):
    n, h = x.shape

    def body(x_ref, deg_ref, xs_ref, dinv_ref):
        d = deg_ref[...]
        dinv = jnp.where(d > 0, lax.rsqrt(d), 0.0)
        dinv_ref[...] = dinv
        xv = x_ref[...]
        xs_ref[0:n] = xv * dinv[0:n]
        xs_ref[n:] = xv * dinv[n:]

    return pl.pallas_call(
        body,
        out_shape=[
            jax.ShapeDtypeStruct((2 * n, h), jnp.float32),
            jax.ShapeDtypeStruct((2 * n, 1), jnp.float32),
        ],
    )(x, deg)


def _tc_final(agg, dinv, x, loop_rel, rel_embed, w_loop, w_in, w_out, w_rel,
              gamma, beta):
    n, h = x.shape
    nr = rel_embed.shape[0]

    def body(agg_ref, dinv_ref, x_ref, loop_ref, re_ref, wl_ref, wi_ref,
             wo_ref, wr_ref, g_ref, b_ref, out_ref, ro_ref):
        dv = dinv_ref[...]
        a_in = agg_ref[0:n] * dv[0:n]
        a_out = agg_ref[n:] * dv[n:]
        t = jnp.dot(a_in, wi_ref[...], preferred_element_type=jnp.float32)
        t = t + jnp.dot(a_out, wo_ref[...], preferred_element_type=jnp.float32)
        t = t + jnp.dot(x_ref[...] * loop_ref[...], wl_ref[...],
                        preferred_element_type=jnp.float32)
        t = t * (1.0 / 3.0)
        mean = jnp.mean(t, axis=0, keepdims=True)
        cen = t - mean
        var = jnp.mean(cen * cen, axis=0, keepdims=True)
        out_ref[...] = g_ref[...] * cen * lax.rsqrt(var + 1e-5) + b_ref[...]
        ro_ref[...] = jnp.dot(re_ref[...], wr_ref[...],
                              preferred_element_type=jnp.float32)

    return pl.pallas_call(
        body,
        out_shape=[
            jax.ShapeDtypeStruct((n, h), jnp.float32),
            jax.ShapeDtypeStruct((nr, h), jnp.float32),
        ],
    )(agg, dinv, x, loop_rel, rel_embed, w_loop, w_in, w_out, w_rel,
      gamma, beta)


def kernel(x, edge_index, edge_type, rel_embed, w_loop, w_in, w_out, w_rel,
           loop_rel, bn_gamma, bn_beta):
    n, h = x.shape
    e2 = edge_index.shape[1]
    e = e2 // 2

    row2 = edge_index[0].astype(jnp.int32)
    col = edge_index[1].astype(jnp.int32)
    colx2 = jnp.concatenate([col[:e], col[e:] + n])
    et2 = edge_type.astype(jnp.int32)
    rel_cat = jnp.concatenate([rel_embed, loop_rel], axis=0)

    deg2 = _sc_degree(row2, n)
    xs_cat, dinv = _tc_scale(x, deg2.reshape(2 * n, 1))
    agg2 = _sc_edges(row2, colx2, et2, xs_cat, rel_cat)
    out, rel_out = _tc_final(agg2, dinv, x, loop_rel, rel_embed, w_loop,
                             w_in, w_out, w_rel,
                             bn_gamma.reshape(1, h), bn_beta.reshape(1, h))
    return out, rel_out


# speedup vs baseline: 17.8021x; 1.7342x over previous
import functools

import jax
import jax.numpy as jnp
from jax import lax
from jax.experimental import pallas as pl
from jax.experimental.pallas import tpu as pltpu
from jax.experimental.pallas import tpu_sc as plsc

_NC = 2
_NS = 16
_L = 16

_MESH = dict(core_axis_name="c", subcore_axis_name="s")


def _sc_degree(row4, n_nodes):
    _, _, g, c = row4.shape
    zchunk = 1000
    depth = 8

    @functools.partial(
        pl.kernel,
        out_type=jax.ShapeDtypeStruct((2 * n_nodes,), jnp.float32),
        mesh=plsc.VectorSubcoreMesh(**_MESH),
        scratch_types=[
            pltpu.VMEM((c,), jnp.float32),
            pltpu.VMEM((g, c), jnp.int32),
            pltpu.VMEM((1024,), jnp.float32),
            pltpu.VMEM_SHARED((n_nodes,), jnp.float32),
            pltpu.SemaphoreType.DMA,
        ],
    )
    def deg_kernel(row_hbm, deg_hbm, ones_v, row_m, zero_v, deg_sh, sem):
        core = lax.axis_index("c")
        sub = lax.axis_index("s")
        for i in range(c // _L):
            ones_v[pl.ds(i * _L, _L)] = jnp.ones((_L,), jnp.float32)
        for i in range(1024 // _L):
            zero_v[pl.ds(i * _L, _L)] = jnp.zeros((_L,), jnp.float32)
        pltpu.sync_copy(row_hbm.at[core, sub], row_m)

        @pl.when(sub < n_nodes // zchunk)
        def _():
            pltpu.sync_copy(zero_v.at[pl.ds(0, zchunk)],
                            deg_sh.at[pl.ds(sub * zchunk, zchunk)])

        plsc.subcore_barrier()

        def fire(i):
            pltpu.async_copy(ones_v, deg_sh.at[row_m.at[i]], sem, add=True)

        def drain(i):
            pltpu.make_async_copy(ones_v, deg_sh.at[row_m.at[i]], sem).wait()

        def chunk(i, carry):
            fire(i)

            @pl.when(i >= depth)
            def _():
                drain(i - depth)

            return carry

        lax.fori_loop(0, g, chunk, 0)
        lax.fori_loop(g - depth, g, lambda i, cy: (drain(i), cy)[1], 0)
        plsc.subcore_barrier()

        @pl.when(sub < n_nodes // zchunk)
        def _():
            pltpu.sync_copy(deg_sh.at[pl.ds(sub * zchunk, zchunk)],
                            zero_v.at[pl.ds(0, zchunk)])
            pltpu.sync_copy(
                zero_v.at[pl.ds(0, zchunk)],
                deg_hbm.at[pl.ds(core * n_nodes + sub * zchunk, zchunk)])

    return deg_kernel(row4)


def _sc_edges(row2, colx2, et2, xs_cat, rel_cat):
    e2 = row2.shape[0]
    e = e2 // 2
    ept = e // _NS
    c = 80
    g = ept // c
    assert g % 2 == 1
    n2, h = xs_cat.shape
    n = n2 // 2
    hl = h // _L
    br = 16
    nb = n // br

    @functools.partial(
        pl.kernel,
        out_type=jax.ShapeDtypeStruct((2 * n, h), jnp.float32),
        mesh=plsc.VectorSubcoreMesh(**_MESH),
        scratch_types=[
            pltpu.VMEM((c,), jnp.int32),
            pltpu.VMEM((c,), jnp.int32),
            pltpu.VMEM((c,), jnp.int32),
            pltpu.VMEM((c,), jnp.int32),
            pltpu.VMEM((c,), jnp.int32),
            pltpu.VMEM((c,), jnp.int32),
            pltpu.VMEM((c, h), jnp.float32),
            pltpu.VMEM((c, h), jnp.float32),
            pltpu.VMEM((c, h), jnp.float32),
            pltpu.VMEM((c, h), jnp.float32),
            pltpu.VMEM((br, h), jnp.float32),
            pltpu.VMEM_SHARED((n, h), jnp.float32),
            pltpu.SemaphoreType.DMA,
            pltpu.SemaphoreType.DMA,
            pltpu.SemaphoreType.DMA,
            pltpu.SemaphoreType.DMA,
            pltpu.SemaphoreType.DMA,
            pltpu.SemaphoreType.DMA,
        ],
    )
    def edge_kernel(row_hbm, col_hbm, et_hbm, xs_hbm, rel_hbm, agg_hbm,
                    rwa, cla, eta, rwb, clb, etb, xba, rba, xbb, rbb,
                    zb, agg_sh, sxa, sra, sxb, srb, sia, sib):
        core = lax.axis_index("c")
        sub = lax.axis_index("s")

        def zrow(r, carry):
            for j in range(hl):
                zb[r, pl.ds(j * _L, _L)] = jnp.zeros((_L,), jnp.float32)
            return carry

        lax.fori_loop(0, br, zrow, 0)

        def zblk(k, carry):
            blk = k * _NS + sub
            @pl.when(blk < nb)
            def _():
                pltpu.sync_copy(zb, agg_sh.at[pl.ds(blk * br, br)])
            return carry

        lax.fori_loop(0, (nb + _NS - 1) // _NS, zblk, 0)
        plsc.subcore_barrier()

        ebase = core * e + sub * ept

        def fire_idx(i, rw, cl, et, si):
            pltpu.async_copy(row_hbm.at[pl.ds(ebase + i * c, c)], rw, si)
            pltpu.async_copy(col_hbm.at[pl.ds(ebase + i * c, c)], cl, si)
            pltpu.async_copy(et_hbm.at[pl.ds(ebase + i * c, c)], et, si)

        def wait_idx(i, rw, cl, et, si):
            pltpu.make_async_copy(row_hbm.at[pl.ds(ebase + i * c, c)], rw,
                                  si).wait()
            pltpu.make_async_copy(col_hbm.at[pl.ds(ebase + i * c, c)], cl,
                                  si).wait()
            pltpu.make_async_copy(et_hbm.at[pl.ds(ebase + i * c, c)], et,
                                  si).wait()

        def fire_gather(cl, et, xb, rb, sx, sr):
            pltpu.async_copy(xs_hbm.at[cl], xb, sx)
            pltpu.async_copy(rel_hbm.at[et], rb, sr)

        def wait_gather(cl, et, xb, rb, sx, sr):
            pltpu.make_async_copy(xs_hbm.at[cl], xb, sx).wait()
            pltpu.make_async_copy(rel_hbm.at[et], rb, sr).wait()

        def compute(rw, xb, rb):
            def prod(k, carry2):
                for j in range(hl):
                    sl = pl.ds(j * _L, _L)
                    xb[k, sl] = xb[k, sl] * rb[k, sl]
                return carry2

            lax.fori_loop(0, c, prod, 0)
            pltpu.sync_copy(xb, agg_sh.at[rw], add=True)

        pltpu.sync_copy(row_hbm.at[pl.ds(ebase, c)], rwa)
        pltpu.sync_copy(col_hbm.at[pl.ds(ebase, c)], cla)
        pltpu.sync_copy(et_hbm.at[pl.ds(ebase, c)], eta)
        fire_gather(cla, eta, xba, rba, sxa, sra)
        fire_idx(1, rwb, clb, etb, sib)

        def pipe(gg, carry):
            a = 2 * gg
            wait_idx(a + 1, rwb, clb, etb, sib)
            fire_gather(clb, etb, xbb, rbb, sxb, srb)
            wait_gather(cla, eta, xba, rba, sxa, sra)
            compute(rwa, xba, rba)

            @pl.when(a + 2 < g)
            def _():
                fire_idx(a + 2, rwa, cla, eta, sia)

            wait_gather(clb, etb, xbb, rbb, sxb, srb)
            compute(rwb, xbb, rbb)

            @pl.when(a + 2 < g)
            def _():
                wait_idx(a + 2, rwa, cla, eta, sia)
                fire_gather(cla, eta, xba, rba, sxa, sra)

            @pl.when(a + 3 < g)
            def _():
                fire_idx(a + 3, rwb, clb, etb, sib)

            return carry

        lax.fori_loop(0, g // 2, pipe, 0)
        wait_gather(cla, eta, xba, rba, sxa, sra)
        compute(rwa, xba, rba)
        plsc.subcore_barrier()

        def wblk(k, carry):
            blk = k * _NS + sub
            @pl.when(blk < nb)
            def _():
                pltpu.sync_copy(agg_sh.at[pl.ds(blk * br, br)], zb)
                pltpu.sync_copy(zb, agg_hbm.at[pl.ds(core * n + blk * br, br)])
            return carry

        lax.fori_loop(0, (nb + _NS - 1) // _NS, wblk, 0)

    return edge_kernel(row2, colx2, et2, xs_cat, rel_cat)


def _tc_scale(x, deg):
    n, h = x.shape

    def body(x_ref, deg_ref, xs_ref, dinv_ref):
        d = deg_ref[...]
        dinv = jnp.where(d > 0, lax.rsqrt(d), 0.0)
        dinv_ref[...] = dinv
        xv = x_ref[...]
        xs_ref[0:n] = xv * dinv[0:n]
        xs_ref[n:] = xv * dinv[n:]

    return pl.pallas_call(
        body,
        out_shape=[
            jax.ShapeDtypeStruct((2 * n, h), jnp.float32),
            jax.ShapeDtypeStruct((2 * n, 1), jnp.float32),
        ],
    )(x, deg)


def _tc_final(agg, dinv, x, loop_rel, rel_embed, w_loop, w_in, w_out, w_rel,
              gamma, beta):
    n, h = x.shape
    nr = rel_embed.shape[0]

    def body(agg_ref, dinv_ref, x_ref, loop_ref, re_ref, wl_ref, wi_ref,
             wo_ref, wr_ref, g_ref, b_ref, out_ref, ro_ref):
        dv = dinv_ref[...]
        a_in = agg_ref[0:n] * dv[0:n]
        a_out = agg_ref[n:] * dv[n:]
        t = jnp.dot(a_in, wi_ref[...], preferred_element_type=jnp.float32)
        t = t + jnp.dot(a_out, wo_ref[...], preferred_element_type=jnp.float32)
        t = t + jnp.dot(x_ref[...] * loop_ref[...], wl_ref[...],
                        preferred_element_type=jnp.float32)
        t = t * (1.0 / 3.0)
        mean = jnp.mean(t, axis=0, keepdims=True)
        cen = t - mean
        var = jnp.mean(cen * cen, axis=0, keepdims=True)
        out_ref[...] = g_ref[...] * cen * lax.rsqrt(var + 1e-5) + b_ref[...]
        ro_ref[...] = jnp.dot(re_ref[...], wr_ref[...],
                              preferred_element_type=jnp.float32)

    return pl.pallas_call(
        body,
        out_shape=[
            jax.ShapeDtypeStruct((n, h), jnp.float32),
            jax.ShapeDtypeStruct((nr, h), jnp.float32),
        ],
    )(agg, dinv, x, loop_rel, rel_embed, w_loop, w_in, w_out, w_rel,
      gamma, beta)


def kernel(x, edge_index, edge_type, rel_embed, w_loop, w_in, w_out, w_rel,
           loop_rel, bn_gamma, bn_beta):
    n, h = x.shape
    e2 = edge_index.shape[1]
    e = e2 // 2

    chunk = 80
    ept = e // _NS

    row2 = edge_index[0].astype(jnp.int32)
    row4 = row2.reshape(2, _NS, ept // chunk, chunk)
    col = edge_index[1].astype(jnp.int32)
    colx2 = jnp.concatenate([col[:e], col[e:] + n])
    et2 = edge_type.astype(jnp.int32)
    rel_cat = jnp.concatenate([rel_embed, loop_rel], axis=0)

    deg2 = _sc_degree(row4, n)
    xs_cat, dinv = _tc_scale(x, deg2.reshape(2 * n, 1))
    agg2 = _sc_edges(row2, colx2, et2, xs_cat, rel_cat)
    out, rel_out = _tc_final(agg2, dinv, x, loop_rel, rel_embed, w_loop,
                             w_in, w_out, w_rel,
                             bn_gamma.reshape(1, h), bn_beta.reshape(1, h))
    return out, rel_out
